# Initial kernel scaffold; baseline (speedup 1.0000x reference)
import jax
import jax.numpy as jnp
from jax.experimental import pallas as pl


def kernel(x, edge_index, batch, emb1, emb2, emb3, emb4, W1a, b1a, W1b, b1b, W2a, b2a, W2b, b2b, W3a, b3a, W3b, b3b, g1, be1, m1, v1, g2, be2, m2, v2, g3, be3, m3, v3, Wl2, bl2):
    raise NotImplementedError("write your pallas kernel here")


# speedup vs baseline: 6.6346x; 6.6346x over previous
import functools
import jax
import jax.numpy as jnp
from jax import lax
from jax.experimental import pallas as pl
from jax.experimental.pallas import tpu as pltpu
from jax.experimental.pallas import tpu_sc as plsc

F32 = jnp.float32
I32 = jnp.int32

NNODES = 50000
NEDGES = 800000
NSEG = 512
D = 64
NP = 51200
PADROWS = NP - NNODES
HALF = NNODES // 2
BLK = 1024
NBLK = NP // BLK

NTILES = 16
NCORES = 2
EPT = 51200
EPAD = EPT * NTILES
SUP = 2048
NSUP = EPT // SUP
K = 128
KPS = SUP // K


def _embed_body(xt_ref, tcat_ref, w5_ref, out_ref):
    i = pl.program_id(0)
    c0 = xt_ref[0, :]
    c1 = xt_ref[1, :]
    c2 = xt_ref[2, :]
    c3 = xt_ref[3, :]
    c4 = xt_ref[4, :]
    i40 = lax.broadcasted_iota(I32, (BLK, 40), 1)
    oh = ((c0[:, None] == i40).astype(F32)
          + ((c1[:, None] + 16) == i40).astype(F32)
          + ((c2[:, None] + 22) == i40).astype(F32)
          + ((c3[:, None] + 28) == i40).astype(F32))
    u = jnp.dot(oh, tcat_ref[...], preferred_element_type=F32)
    u = u + (c4.astype(F32) - 1.0)[:, None] * w5_ref[0, :][None, :]
    row = i * BLK + lax.broadcasted_iota(I32, (BLK, D), 0)
    out_ref[...] = jnp.where(row < NNODES, u, 0.0)


def _embed(xt, tcat, w5):
    return pl.pallas_call(
        _embed_body,
        grid=(NBLK,),
        in_specs=[
            pl.BlockSpec((8, BLK), lambda i: (0, i)),
            pl.BlockSpec((40, D), lambda i: (0, 0)),
            pl.BlockSpec((1, D), lambda i: (0, 0)),
        ],
        out_specs=pl.BlockSpec((BLK, D), lambda i: (i, 0)),
        out_shape=jax.ShapeDtypeStruct((NP, D), F32),
    )(xt, tcat, w5)


def _dense1_body(z_ref, ba_ref, wb_ref, bb_ref, s_ref, t_ref, out_ref):
    i = pl.program_id(0)
    h = jnp.maximum(z_ref[...] + ba_ref[0, :][None, :], 0.0)
    y = jnp.dot(h, wb_ref[...], preferred_element_type=F32) + bb_ref[0, :][None, :]
    y = jnp.maximum(y, 0.0) * s_ref[0, :][None, :] + t_ref[0, :][None, :]
    row = i * BLK + lax.broadcasted_iota(I32, (BLK, D), 0)
    out_ref[...] = jnp.where(row < NNODES, y, 0.0)


def _dense1(z, ba, wb, bb, s, t):
    vec = pl.BlockSpec((1, D), lambda i: (0, 0))
    return pl.pallas_call(
        _dense1_body,
        grid=(NBLK,),
        in_specs=[pl.BlockSpec((BLK, D), lambda i: (i, 0)), vec,
                  pl.BlockSpec((D, D), lambda i: (0, 0)), vec, vec, vec],
        out_specs=pl.BlockSpec((BLK, D), lambda i: (i, 0)),
        out_shape=jax.ShapeDtypeStruct((NP, D), F32),
    )(z, ba, wb, bb, s, t)


def _dense2_body(z_ref, wa_ref, ba_ref, wb_ref, bb_ref, s_ref, t_ref, out_ref):
    i = pl.program_id(0)
    h = jnp.dot(z_ref[...], wa_ref[...], preferred_element_type=F32)
    h = jnp.maximum(h + ba_ref[0, :][None, :], 0.0)
    y = jnp.dot(h, wb_ref[...], preferred_element_type=F32) + bb_ref[0, :][None, :]
    y = jnp.maximum(y, 0.0) * s_ref[0, :][None, :] + t_ref[0, :][None, :]
    row = i * BLK + lax.broadcasted_iota(I32, (BLK, D), 0)
    out_ref[...] = jnp.where(row < NNODES, y, 0.0)


def _dense2(z, wa, ba, wb, bb, s, t):
    H = wa.shape[1]
    vh = pl.BlockSpec((1, H), lambda i: (0, 0))
    vd = pl.BlockSpec((1, D), lambda i: (0, 0))
    return pl.pallas_call(
        _dense2_body,
        grid=(NBLK,),
        in_specs=[pl.BlockSpec((BLK, D), lambda i: (i, 0)),
                  pl.BlockSpec((D, H), lambda i: (0, 0)), vh,
                  pl.BlockSpec((H, D), lambda i: (0, 0)), vd, vd, vd],
        out_specs=pl.BlockSpec((BLK, D), lambda i: (i, 0)),
        out_shape=jax.ShapeDtypeStruct((NP, D), F32),
    )(z, wa, ba, wb, bb, s, t)


def _segsum_body(h_hbm, src_hbm, dst_hbm, z_hbm,
                 agg, sblk, dblk, gsrc, gdst, rowbuf,
                 sem_in, sem_g, sem_s):
    c = lax.axis_index("c")
    s = lax.axis_index("s")
    base = c * HALF

    rpt = 1562
    pltpu.sync_copy(h_hbm.at[pl.ds(base + s * rpt, rpt), :],
                    agg.at[pl.ds(s * rpt, rpt), :])

    @pl.when(s == NTILES - 1)
    def _init_tail():
        pltpu.sync_copy(h_hbm.at[pl.ds(base + 24992, 8), :],
                        agg.at[pl.ds(24992, 8), :])

    plsc.subcore_barrier()

    eb = s * EPT

    def super_chunk(c0, _):
        off = eb + c0 * SUP
        pltpu.async_copy(src_hbm.at[pl.ds(off, SUP)], sblk, sem_in).wait()
        pltpu.async_copy(dst_hbm.at[pl.ds(off, SUP)], dblk, sem_in).wait()
        for j in range(KPS):
            for k in range(8):
                o = j * K + k * 16
                s16 = sblk[pl.ds(o, 16)]
                d16 = dblk[pl.ds(o, 16)]
                lane = lax.iota(I32, 16) + (off + o)
                owned = (d16 >= base) & (d16 < base + HALF)
                gs = jnp.where(owned, s16, NNODES + (lane & 1023))
                gd = jnp.where(owned, d16 - base, lane & 16383)
                gsrc[j, 0, pl.ds(k * 16, 16)] = gs
                gdst[j, 0, pl.ds(k * 16, 16)] = gd
        for j in range(KPS):
            gcp = pltpu.async_copy(h_hbm.at[gsrc.at[j, 0]],
                                   rowbuf.at[j % 2], sem_g)
            if j > 0:
                pltpu.async_copy(rowbuf.at[(j - 1) % 2], agg.at[gdst.at[j - 1, 0]],
                                 sem_s, add=True).wait()
            gcp.wait()
        pltpu.async_copy(rowbuf.at[(KPS - 1) % 2], agg.at[gdst.at[KPS - 1, 0]],
                         sem_s, add=True).wait()
        return 0

    lax.fori_loop(0, NSUP, super_chunk, 0, unroll=False)

    plsc.subcore_barrier()
    pltpu.sync_copy(agg.at[pl.ds(s * rpt, rpt), :],
                    z_hbm.at[pl.ds(base + s * rpt, rpt), :])

    @pl.when(s == NTILES - 1)
    def _wb_tail():
        pltpu.sync_copy(agg.at[pl.ds(24992, 8), :],
                        z_hbm.at[pl.ds(base + 24992, 8), :])


def _segsum(h, src, dst):
    mesh = plsc.VectorSubcoreMesh(core_axis_name="c", subcore_axis_name="s")
    fn = pl.kernel(
        _segsum_body,
        out_type=jax.ShapeDtypeStruct((NP, D), F32),
        mesh=mesh,
        scratch_types=[
            pltpu.VMEM_SHARED((HALF, D), F32),
            pltpu.VMEM((SUP,), I32),
            pltpu.VMEM((SUP,), I32),
            pltpu.VMEM((KPS, 1, K), I32),
            pltpu.VMEM((KPS, 1, K), I32),
            pltpu.VMEM((2, K, D), F32),
            pltpu.SemaphoreType.DMA,
            pltpu.SemaphoreType.DMA,
            pltpu.SemaphoreType.DMA,
        ],
        compiler_params=pltpu.CompilerParams(use_tc_tiling_on_sc=False),
    )
    return fn(h, src, dst)


RPT = 1560
TAIL = NNODES - 32 * RPT
BSTAGE = 1656
CHUNK = 512


def _segmax_body(x_hbm, b_hbm, part_hbm, xbuf, bbuf, pbuf, sem):
    c = lax.axis_index("c")
    s = lax.axis_index("s")
    w = c * NTILES + s
    r0 = w * RPT
    nrows = RPT + jnp.where(w == 31, TAIL, 0)

    neg = jnp.full((16,), -jnp.inf, F32)

    def init_row(i, _):
        for kk in range(4):
            pbuf[i, pl.ds(kk * 16, 16)] = neg
        return 0
    lax.fori_loop(0, NSEG, init_row, 0)

    pltpu.async_copy(b_hbm.at[pl.ds(r0, BSTAGE)], bbuf, sem).wait()

    def chunk_loop(ci, _):
        nhere = jnp.minimum(nrows - ci * CHUNK, CHUNK)
        pltpu.async_copy(x_hbm.at[pl.ds(r0 + ci * CHUNK, CHUNK), :], xbuf,
                         sem).wait()

        def row_loop(i, _):
            b = bbuf[pl.ds(ci * CHUNK + i, 16)][0]
            for kk in range(4):
                sl = pl.ds(kk * 16, 16)
                pbuf[b, sl] = jnp.maximum(pbuf[b, sl], xbuf[i, sl])
            return 0

        lax.fori_loop(0, nhere, row_loop, 0)
        return 0

    lax.fori_loop(0, 4, chunk_loop, 0)

    pltpu.sync_copy(pbuf, part_hbm.at[w])


def _segmax(x3, batch_pad):
    mesh = plsc.VectorSubcoreMesh(core_axis_name="c", subcore_axis_name="s")
    fn = pl.kernel(
        _segmax_body,
        out_type=jax.ShapeDtypeStruct((32, NSEG, D), F32),
        mesh=mesh,
        scratch_types=[
            pltpu.VMEM((CHUNK, D), F32),
            pltpu.VMEM((BSTAGE,), I32),
            pltpu.VMEM((NSEG, D), F32),
            pltpu.SemaphoreType.DMA,
        ],
        compiler_params=pltpu.CompilerParams(use_tc_tiling_on_sc=False),
    )
    return fn(x3, batch_pad)


def _final_body(p_ref, wl_ref, bl_ref, out_ref):
    p = p_ref[...]
    m = jnp.max(p, axis=0)
    out_ref[...] = jnp.dot(m, wl_ref[...], preferred_element_type=F32) \
        + bl_ref[0, :][None, :]


def _final(partials, wl2, bl2):
    return pl.pallas_call(
        _final_body,
        in_specs=[pl.BlockSpec((32, NSEG, D), lambda: (0, 0, 0)),
                  pl.BlockSpec((D, 128), lambda: (0, 0)),
                  pl.BlockSpec((1, 128), lambda: (0, 0))],
        out_specs=pl.BlockSpec((NSEG, 128), lambda: (0, 0)),
        out_shape=jax.ShapeDtypeStruct((NSEG, 128), F32),
    )(partials, wl2, bl2)


def kernel(x, edge_index, batch, emb1, emb2, emb3, emb4,
           W1a, b1a, W1b, b1b, W2a, b2a, W2b, b2b, W3a, b3a, W3b, b3b,
           g1, be1, m1, v1, g2, be2, m2, v2, g3, be3, m3, v3, Wl2, bl2):
    tcat = jnp.concatenate([
        emb1 @ W1a[:32], emb2 @ W1a[32:64], emb3 @ W1a[32:64],
        emb4 @ W1a[32:64], jnp.zeros((6, D), F32)], axis=0)
    w5 = W1a[64].reshape(1, D)

    def bnfold(g, be, m, v):
        sc = g / jnp.sqrt(v + 1e-5)
        return sc.reshape(1, D), (be - m * sc).reshape(1, D)

    s1, t1 = bnfold(g1, be1, m1, v1)
    s2, t2 = bnfold(g2, be2, m2, v2)
    s3, t3 = bnfold(g3, be3, m3, v3)

    xt = jnp.pad(x, ((0, NP - NNODES), (0, 0))).T
    npad = EPAD - NEDGES
    src = jnp.concatenate([edge_index[0],
                           NNODES + (jnp.arange(npad, dtype=I32) & 1023)])
    dst = jnp.concatenate([edge_index[1],
                           jnp.full((npad,), NP, I32)])
    batch_pad = jnp.pad(batch, (0, 31 * RPT + BSTAGE - NNODES),
                        constant_values=NSEG - 1)

    u = _embed(xt, tcat, w5)
    z1 = _segsum(u, src, dst)
    x1 = _dense1(z1, b1a.reshape(1, D), W1b, b1b.reshape(1, D), s1, t1)
    z2 = _segsum(x1, src, dst)
    x2 = _dense2(z2, W2a, b2a.reshape(1, 256), W2b, b2b.reshape(1, D), s2, t2)
    z3 = _segsum(x2, src, dst)
    x3 = _dense2(z3, W3a, b3a.reshape(1, 256), W3b, b3b.reshape(1, D), s3, t3)
    partials = _segmax(x3, batch_pad)
    return _final(partials, Wl2, bl2.reshape(1, 128))


# speedup vs baseline: 11.3551x; 1.7115x over previous
import jax
import jax.numpy as jnp
from jax import lax
from jax.experimental import pallas as pl
from jax.experimental.pallas import tpu as pltpu
from jax.experimental.pallas import tpu_sc as plsc

F32 = jnp.float32
I32 = jnp.int32

NNODES = 50000
NEDGES = 800000
NSEG = 512
D = 64
NP = 51200
HALF = NNODES // 2
BLK = 1024
NBLK = NP // BLK

NTILES = 16
NCORES = 2
EPAD = 819200
EPW = EPAD // 32
PSUP = 3200
FLUSH = 2048
CAP = 28672
DUMP = FLUSH + 16
K = 128
KPS = FLUSH // K
NBUF = 2
AHEAD = 1


def _embed_body(xt_ref, tcat_ref, w5_ref, out_ref):
    i = pl.program_id(0)
    c0 = xt_ref[0, :]
    c1 = xt_ref[1, :]
    c2 = xt_ref[2, :]
    c3 = xt_ref[3, :]
    c4 = xt_ref[4, :]
    i40 = lax.broadcasted_iota(I32, (BLK, 40), 1)
    oh = ((c0[:, None] == i40).astype(F32)
          + ((c1[:, None] + 16) == i40).astype(F32)
          + ((c2[:, None] + 22) == i40).astype(F32)
          + ((c3[:, None] + 28) == i40).astype(F32))
    u = jnp.dot(oh, tcat_ref[...], preferred_element_type=F32)
    u = u + (c4.astype(F32) - 1.0)[:, None] * w5_ref[0, :][None, :]
    row = i * BLK + lax.broadcasted_iota(I32, (BLK, D), 0)
    out_ref[...] = jnp.where(row < NNODES, u, 0.0)


def _embed(xt, tcat, w5):
    return pl.pallas_call(
        _embed_body,
        grid=(NBLK,),
        in_specs=[
            pl.BlockSpec((8, BLK), lambda i: (0, i)),
            pl.BlockSpec((40, D), lambda i: (0, 0)),
            pl.BlockSpec((1, D), lambda i: (0, 0)),
        ],
        out_specs=pl.BlockSpec((BLK, D), lambda i: (i, 0)),
        out_shape=jax.ShapeDtypeStruct((NP, D), F32),
    )(xt, tcat, w5)


def _dense1_body(z_ref, ba_ref, wb_ref, bb_ref, s_ref, t_ref, out_ref):
    i = pl.program_id(0)
    h = jnp.maximum(z_ref[...] + ba_ref[0, :][None, :], 0.0)
    y = jnp.dot(h, wb_ref[...], preferred_element_type=F32) + bb_ref[0, :][None, :]
    y = jnp.maximum(y, 0.0) * s_ref[0, :][None, :] + t_ref[0, :][None, :]
    row = i * BLK + lax.broadcasted_iota(I32, (BLK, D), 0)
    out_ref[...] = jnp.where(row < NNODES, y, 0.0)


def _dense1(z, ba, wb, bb, s, t):
    vec = pl.BlockSpec((1, D), lambda i: (0, 0))
    return pl.pallas_call(
        _dense1_body,
        grid=(NBLK,),
        in_specs=[pl.BlockSpec((BLK, D), lambda i: (i, 0)), vec,
                  pl.BlockSpec((D, D), lambda i: (0, 0)), vec, vec, vec],
        out_specs=pl.BlockSpec((BLK, D), lambda i: (i, 0)),
        out_shape=jax.ShapeDtypeStruct((NP, D), F32),
    )(z, ba, wb, bb, s, t)


def _dense2_body(z_ref, wa_ref, ba_ref, wb_ref, bb_ref, s_ref, t_ref, out_ref):
    i = pl.program_id(0)
    h = jnp.dot(z_ref[...], wa_ref[...], preferred_element_type=F32)
    h = jnp.maximum(h + ba_ref[0, :][None, :], 0.0)
    y = jnp.dot(h, wb_ref[...], preferred_element_type=F32) + bb_ref[0, :][None, :]
    y = jnp.maximum(y, 0.0) * s_ref[0, :][None, :] + t_ref[0, :][None, :]
    row = i * BLK + lax.broadcasted_iota(I32, (BLK, D), 0)
    out_ref[...] = jnp.where(row < NNODES, y, 0.0)


def _dense2(z, wa, ba, wb, bb, s, t):
    H = wa.shape[1]
    vh = pl.BlockSpec((1, H), lambda i: (0, 0))
    vd = pl.BlockSpec((1, D), lambda i: (0, 0))
    return pl.pallas_call(
        _dense2_body,
        grid=(NBLK,),
        in_specs=[pl.BlockSpec((BLK, D), lambda i: (i, 0)),
                  pl.BlockSpec((D, H), lambda i: (0, 0)), vh,
                  pl.BlockSpec((H, D), lambda i: (0, 0)), vd, vd, vd],
        out_specs=pl.BlockSpec((BLK, D), lambda i: (i, 0)),
        out_shape=jax.ShapeDtypeStruct((NP, D), F32),
    )(z, wa, ba, wb, bb, s, t)


def _part_body(src_hbm, dst_hbm, bsrc_hbm, bdst_hbm, cnt_hbm,
               sblk, dblk, f0s, f0d, f1s, f1d, cbuf, sem):
    c = lax.axis_index("c")
    s = lax.axis_index("s")
    w = c * NTILES + s
    eb = w * EPW
    i16 = lax.iota(I32, 16)

    def vstep(k, carry):
        o0, f0, o1, f1 = carry
        s16 = sblk[pl.ds(k * 16, 16)]
        d16 = dblk[pl.ds(k * 16, 16)]
        m0 = d16 < HALF
        m1 = (d16 >= HALF) & (d16 < NNODES)
        cm0 = plsc.cumsum(m0.astype(I32))
        cm1 = plsc.cumsum(m1.astype(I32))
        pos0 = jnp.where(m0, o0 + cm0 - 1, DUMP + i16)
        pos1 = jnp.where(m1, o1 + cm1 - 1, DUMP + i16)
        plsc.store_scatter(f0s, [pos0], s16)
        plsc.store_scatter(f0d, [pos0], d16)
        plsc.store_scatter(f1s, [pos1], s16)
        plsc.store_scatter(f1d, [pos1], d16 - HALF)
        o0n = o0 + cm0[15]
        o1n = o1 + cm1[15]

        @pl.when(o0n >= FLUSH)
        def _fl0():
            pltpu.sync_copy(f0s.at[pl.ds(0, FLUSH)],
                            bsrc_hbm.at[0, w, pl.ds(f0 * FLUSH, FLUSH)])
            pltpu.sync_copy(f0d.at[pl.ds(0, FLUSH)],
                            bdst_hbm.at[0, w, pl.ds(f0 * FLUSH, FLUSH)])
            f0s[pl.ds(0, 16)] = f0s[pl.ds(FLUSH, 16)]
            f0d[pl.ds(0, 16)] = f0d[pl.ds(FLUSH, 16)]

        @pl.when(o1n >= FLUSH)
        def _fl1():
            pltpu.sync_copy(f1s.at[pl.ds(0, FLUSH)],
                            bsrc_hbm.at[1, w, pl.ds(f1 * FLUSH, FLUSH)])
            pltpu.sync_copy(f1d.at[pl.ds(0, FLUSH)],
                            bdst_hbm.at[1, w, pl.ds(f1 * FLUSH, FLUSH)])
            f1s[pl.ds(0, 16)] = f1s[pl.ds(FLUSH, 16)]
            f1d[pl.ds(0, 16)] = f1d[pl.ds(FLUSH, 16)]

        o0o = jnp.where(o0n >= FLUSH, o0n - FLUSH, o0n)
        f0o = jnp.where(o0n >= FLUSH, f0 + 1, f0)
        o1o = jnp.where(o1n >= FLUSH, o1n - FLUSH, o1n)
        f1o = jnp.where(o1n >= FLUSH, f1 + 1, f1)
        return (o0o, f0o, o1o, f1o)

    def sup(si, carry):
        off = eb + si * PSUP
        pltpu.async_copy(src_hbm.at[pl.ds(off, PSUP)], sblk, sem).wait()
        pltpu.async_copy(dst_hbm.at[pl.ds(off, PSUP)], dblk, sem).wait()
        return lax.fori_loop(0, PSUP // 16, vstep, carry)

    z0 = jnp.int32(0)
    o0, f0, o1, f1 = lax.fori_loop(0, EPW // PSUP, sup, (z0, z0, z0, z0))

    def tailfill(srcbuf, dstbuf, o, salt):
        def fill(i, _):
            off = o + i * 16
            srcbuf[pl.ds(off, 16)] = NNODES + ((off + i16 + salt) & 1023)
            dstbuf[pl.ds(off, 16)] = (off * 7 + i16 * 13 + salt) & 16383
            return 0
        lax.fori_loop(0, (FLUSH - o + 15) // 16, fill, 0)

    @pl.when(o0 > 0)
    def _tf0():
        tailfill(f0s, f0d, o0, w * 29)
        pltpu.sync_copy(f0s.at[pl.ds(0, FLUSH)],
                        bsrc_hbm.at[0, w, pl.ds(f0 * FLUSH, FLUSH)])
        pltpu.sync_copy(f0d.at[pl.ds(0, FLUSH)],
                        bdst_hbm.at[0, w, pl.ds(f0 * FLUSH, FLUSH)])

    @pl.when(o1 > 0)
    def _tf1():
        tailfill(f1s, f1d, o1, w * 97)
        pltpu.sync_copy(f1s.at[pl.ds(0, FLUSH)],
                        bsrc_hbm.at[1, w, pl.ds(f1 * FLUSH, FLUSH)])
        pltpu.sync_copy(f1d.at[pl.ds(0, FLUSH)],
                        bdst_hbm.at[1, w, pl.ds(f1 * FLUSH, FLUSH)])

    n0tot = (f0 + jnp.where(o0 > 0, 1, 0)) * FLUSH
    n1tot = (f1 + jnp.where(o1 > 0, 1, 0)) * FLUSH
    cbuf[pl.ds(0, 16)] = jnp.zeros((16,), I32) + n0tot
    pltpu.sync_copy(cbuf, cnt_hbm.at[0, w])
    cbuf[pl.ds(0, 16)] = jnp.zeros((16,), I32) + n1tot
    pltpu.sync_copy(cbuf, cnt_hbm.at[1, w])


def _partition(src, dst):
    mesh = plsc.VectorSubcoreMesh(core_axis_name="c", subcore_axis_name="s")
    fn = pl.kernel(
        _part_body,
        out_type=(jax.ShapeDtypeStruct((2, 32, CAP), I32),
                  jax.ShapeDtypeStruct((2, 32, CAP), I32),
                  jax.ShapeDtypeStruct((2, 32, 16), I32)),
        mesh=mesh,
        scratch_types=[
            pltpu.VMEM((PSUP,), I32),
            pltpu.VMEM((PSUP,), I32),
            pltpu.VMEM((FLUSH + 32,), I32),
            pltpu.VMEM((FLUSH + 32,), I32),
            pltpu.VMEM((FLUSH + 32,), I32),
            pltpu.VMEM((FLUSH + 32,), I32),
            pltpu.VMEM((16,), I32),
            pltpu.SemaphoreType.DMA,
        ],
        compiler_params=pltpu.CompilerParams(use_tc_tiling_on_sc=False,
                                             needs_layout_passes=False),
    )
    return fn(src, dst)


def _segsum_body(h_hbm, bsrc_hbm, bdst_hbm, cnt_hbm, z_hbm,
                 agg, sblk, dblk, gdst, rowbuf, cbuf,
                 sem_in, sem_g, sem_s):
    c = lax.axis_index("c")
    s = lax.axis_index("s")
    base = c * HALF

    rpt = 1562
    pltpu.sync_copy(h_hbm.at[pl.ds(base + s * rpt, rpt), :],
                    agg.at[pl.ds(s * rpt, rpt), :])

    @pl.when(s == NTILES - 1)
    def _init_tail():
        pltpu.sync_copy(h_hbm.at[pl.ds(base + 24992, 8), :],
                        agg.at[pl.ds(24992, 8), :])

    plsc.subcore_barrier()

    for q in range(2):
        t = 2 * s + q
        pltpu.async_copy(cnt_hbm.at[c, t], cbuf, sem_in).wait()
        n = cbuf[pl.ds(0, 16)][0]

        def super_chunk(c0, _):
            off = c0 * FLUSH
            pltpu.async_copy(bsrc_hbm.at[c, t, pl.ds(off, FLUSH)], sblk,
                             sem_in).wait()
            pltpu.async_copy(bdst_hbm.at[c, t, pl.ds(off, FLUSH)], dblk,
                             sem_in).wait()
            for j in range(KPS):
                for k in range(8):
                    gdst[j, 0, pl.ds(k * 16, 16)] = \
                        dblk[pl.ds(j * K + k * 16, 16)]
            gds = [None] * KPS
            sds = [None] * KPS

            def fire_scatter(i):
                gds[i].wait()
                sds[i] = pltpu.async_copy(rowbuf.at[i % NBUF],
                                          agg.at[gdst.at[i, 0]],
                                          sem_s, add=True)

            for j in range(KPS):
                if j >= NBUF:
                    sds[j - NBUF].wait()
                gds[j] = pltpu.async_copy(h_hbm.at[sblk.at[pl.ds(j * K, K)]],
                                          rowbuf.at[j % NBUF], sem_g)
                if j >= AHEAD:
                    fire_scatter(j - AHEAD)
            for i in range(KPS - AHEAD, KPS):
                fire_scatter(i)
            for i in range(KPS - NBUF, KPS):
                sds[i].wait()
            return 0

        lax.fori_loop(0, n // FLUSH, super_chunk, 0)

    plsc.subcore_barrier()
    pltpu.sync_copy(agg.at[pl.ds(s * rpt, rpt), :],
                    z_hbm.at[pl.ds(base + s * rpt, rpt), :])

    @pl.when(s == NTILES - 1)
    def _wb_tail():
        pltpu.sync_copy(agg.at[pl.ds(24992, 8), :],
                        z_hbm.at[pl.ds(base + 24992, 8), :])


def _segsum(h, bsrc, bdst, cnt):
    mesh = plsc.VectorSubcoreMesh(core_axis_name="c", subcore_axis_name="s")
    fn = pl.kernel(
        _segsum_body,
        out_type=jax.ShapeDtypeStruct((NP, D), F32),
        mesh=mesh,
        scratch_types=[
            pltpu.VMEM_SHARED((HALF, D), F32),
            pltpu.VMEM((FLUSH,), I32),
            pltpu.VMEM((FLUSH,), I32),
            pltpu.VMEM((KPS, 1, K), I32),
            pltpu.VMEM((NBUF, K, D), F32),
            pltpu.VMEM((16,), I32),
            pltpu.SemaphoreType.DMA,
            pltpu.SemaphoreType.DMA,
            pltpu.SemaphoreType.DMA,
        ],
        compiler_params=pltpu.CompilerParams(use_tc_tiling_on_sc=False),
    )
    return fn(h, bsrc, bdst, cnt)


RPT = 1560
TAIL = NNODES - 32 * RPT
BSTAGE = 1656
CHUNK = 512


def _segmax_body(x_hbm, b_hbm, part_hbm, xbuf, bbuf, pbuf, sem):
    c = lax.axis_index("c")
    s = lax.axis_index("s")
    w = c * NTILES + s
    r0 = w * RPT
    nrows = RPT + jnp.where(w == 31, TAIL, 0)

    neg = jnp.full((16,), -jnp.inf, F32)

    def init_row(i, _):
        for kk in range(4):
            pbuf[i, pl.ds(kk * 16, 16)] = neg
        return 0
    lax.fori_loop(0, NSEG, init_row, 0)

    pltpu.async_copy(b_hbm.at[pl.ds(r0, BSTAGE)], bbuf, sem).wait()

    def chunk_loop(ci, _):
        nhere = jnp.minimum(nrows - ci * CHUNK, CHUNK)
        pltpu.async_copy(x_hbm.at[pl.ds(r0 + ci * CHUNK, CHUNK), :], xbuf,
                         sem).wait()

        def row_loop(i, _):
            b = bbuf[pl.ds(ci * CHUNK + i, 16)][0]
            for kk in range(4):
                sl = pl.ds(kk * 16, 16)
                pbuf[b, sl] = jnp.maximum(pbuf[b, sl], xbuf[i, sl])
            return 0

        lax.fori_loop(0, nhere, row_loop, 0)
        return 0

    lax.fori_loop(0, 4, chunk_loop, 0)

    pltpu.sync_copy(pbuf, part_hbm.at[w])


def _segmax(x3, batch_pad):
    mesh = plsc.VectorSubcoreMesh(core_axis_name="c", subcore_axis_name="s")
    fn = pl.kernel(
        _segmax_body,
        out_type=jax.ShapeDtypeStruct((32, NSEG, D), F32),
        mesh=mesh,
        scratch_types=[
            pltpu.VMEM((CHUNK, D), F32),
            pltpu.VMEM((BSTAGE,), I32),
            pltpu.VMEM((NSEG, D), F32),
            pltpu.SemaphoreType.DMA,
        ],
        compiler_params=pltpu.CompilerParams(use_tc_tiling_on_sc=False),
    )
    return fn(x3, batch_pad)


def _final_body(p_ref, wl_ref, bl_ref, out_ref):
    p = p_ref[...]
    m = jnp.max(p, axis=0)
    out_ref[...] = jnp.dot(m, wl_ref[...], preferred_element_type=F32) \
        + bl_ref[0, :][None, :]


def _final(partials, wl2, bl2):
    return pl.pallas_call(
        _final_body,
        in_specs=[pl.BlockSpec((32, NSEG, D), lambda: (0, 0, 0)),
                  pl.BlockSpec((D, 128), lambda: (0, 0)),
                  pl.BlockSpec((1, 128), lambda: (0, 0))],
        out_specs=pl.BlockSpec((NSEG, 128), lambda: (0, 0)),
        out_shape=jax.ShapeDtypeStruct((NSEG, 128), F32),
    )(partials, wl2, bl2)


def kernel(x, edge_index, batch, emb1, emb2, emb3, emb4,
           W1a, b1a, W1b, b1b, W2a, b2a, W2b, b2b, W3a, b3a, W3b, b3b,
           g1, be1, m1, v1, g2, be2, m2, v2, g3, be3, m3, v3, Wl2, bl2):
    tcat = jnp.concatenate([
        emb1 @ W1a[:32], emb2 @ W1a[32:64], emb3 @ W1a[32:64],
        emb4 @ W1a[32:64], jnp.zeros((6, D), F32)], axis=0)
    w5 = W1a[64].reshape(1, D)

    def bnfold(g, be, m, v):
        sc = g / jnp.sqrt(v + 1e-5)
        return sc.reshape(1, D), (be - m * sc).reshape(1, D)

    s1, t1 = bnfold(g1, be1, m1, v1)
    s2, t2 = bnfold(g2, be2, m2, v2)
    s3, t3 = bnfold(g3, be3, m3, v3)

    xt = jnp.pad(x, ((0, NP - NNODES), (0, 0))).T
    npad = EPAD - NEDGES
    src = jnp.concatenate([edge_index[0],
                           NNODES + (jnp.arange(npad, dtype=I32) & 1023)])
    dst = jnp.concatenate([edge_index[1],
                           jnp.full((npad,), NP, I32)])
    batch_pad = jnp.pad(batch, (0, 31 * RPT + BSTAGE - NNODES),
                        constant_values=NSEG - 1)

    bsrc, bdst, cnt = _partition(src, dst)

    u = _embed(xt, tcat, w5)
    z1 = _segsum(u, bsrc, bdst, cnt)
    x1 = _dense1(z1, b1a.reshape(1, D), W1b, b1b.reshape(1, D), s1, t1)
    z2 = _segsum(x1, bsrc, bdst, cnt)
    x2 = _dense2(z2, W2a, b2a.reshape(1, 256), W2b, b2b.reshape(1, D), s2, t2)
    z3 = _segsum(x2, bsrc, bdst, cnt)
    x3 = _dense2(z3, W3a, b3a.reshape(1, 256), W3b, b3b.reshape(1, D), s3, t3)
    partials = _segmax(x3, batch_pad)
    return _final(partials, Wl2, bl2.reshape(1, 128))
